# baseline (speedup 1.0000x reference)
import functools

import jax
import jax.numpy as jnp
import numpy as np
from jax import lax
from jax.experimental import pallas as pl
from jax.experimental.pallas import tpu as pltpu
from jax.experimental.pallas import tpu_sc as plsc

N = 10000
E = 320000
D = 128
EPS = 1e-5

CHUNK = 128
HALF = CHUNK // 2
NC = 2
NS = 16
NW = NC * NS
KPW = 80
STAGE = 8
NSTAGE = KPW // STAGE
EPAD = NW * KPW * CHUNK
NPAD = 10112
ROWS_PER_TILE = NPAD // NS

def _sc_spmm_body(x_hbm, col_hbm, row_hbm, w_hbm, out_hbm,
                  col_s, row_s, w_s, fb0, fb1,
                  acc, gs0, gs1, ss0, ss1):
    cid = lax.axis_index("c")
    sid = lax.axis_index("s")
    wid = sid * NC + cid
    wstart = wid * KPW
    fbufs = (fb0, fb1)
    gsems = (gs0, gs1)
    ssems = (ss0, ss1)
    fb = fb0

    zero16 = jnp.zeros((16,), jnp.float32)

    def zrow(r, carry):
        for j in range(D // 16):
            fb[r, pl.ds(16 * j, 16)] = zero16
        return carry

    lax.fori_loop(0, CHUNK, zrow, 0)

    zbase = sid * ROWS_PER_TILE
    for kk in range(4):
        pltpu.sync_copy(fb, acc.at[pl.ds(zbase + CHUNK * kk, CHUNK)])
    remr = ROWS_PER_TILE - 4 * CHUNK
    pltpu.sync_copy(fb.at[pl.ds(0, remr)],
                    acc.at[pl.ds(zbase + 4 * CHUNK, remr)])

    plsc.subcore_barrier()

    def load_stage(st, slot):
        off = wstart + STAGE * st
        pltpu.sync_copy(col_hbm.at[pl.ds(off, STAGE)], col_s.at[slot])
        pltpu.sync_copy(row_hbm.at[pl.ds(off, STAGE)], row_s.at[slot])
        pltpu.sync_copy(w_hbm.at[pl.ds(off, STAGE)], w_s.at[slot])

    def gather_start(k, b):
        slot = lax.rem(lax.div(k, STAGE), 2)
        kp = lax.rem(k, STAGE)
        pltpu.async_copy(x_hbm.at[col_s.at[slot, kp]], fbufs[b], gsems[b])

    def gather_wait(k, b):
        slot = lax.rem(lax.div(k, STAGE), 2)
        kp = lax.rem(k, STAGE)
        pltpu.make_async_copy(
            x_hbm.at[col_s.at[slot, kp]], fbufs[b], gsems[b]).wait()

    def scale_inplace(buf, slot, kp):
        def sgroup(g, carry):
            r0 = 16 * g
            wv = w_s[slot, kp, pl.ds(r0, 16)]
            for rp in range(16):
                wr = wv[rp]
                r = r0 + rp
                for j in range(D // 16):
                    sl = pl.ds(16 * j, 16)
                    buf[r, sl] = buf[r, sl] * wr
            return carry

        lax.fori_loop(0, CHUNK // 16, sgroup, 0)

    def scatter_start(slot, kp, b):
        pltpu.async_copy(fbufs[b], acc.at[row_s.at[slot, kp]],
                         ssems[b], add=True)

    def scatter_wait(slot, kp, b):
        pltpu.make_async_copy(fbufs[b], acc.at[row_s.at[slot, kp]],
                              ssems[b]).wait()

    def chunk_body(k, b):
        slot = lax.rem(lax.div(k, STAGE), 2)
        kp = lax.rem(k, STAGE)
        st = lax.div(k, STAGE)

        @pl.when(jnp.logical_and(kp == 0, st + 1 < NSTAGE))
        def _():
            load_stage(st + 1, lax.rem(st + 1, 2))

        gather_wait(k, b)

        @pl.when(k + 1 < KPW)
        def _():
            @pl.when(k >= 1)
            def _():
                scatter_wait(slot, kp, 1 - b)

            gather_start(k + 1, 1 - b)

        scale_inplace(fbufs[b], slot, kp)
        scatter_start(slot, kp, b)

    load_stage(0, 0)
    gather_start(0, 0)

    def pair_body(i, carry):
        chunk_body(2 * i, 0)
        chunk_body(2 * i + 1, 1)
        return carry

    lax.fori_loop(0, KPW // 2, pair_body, 0)

    last_slot = (NSTAGE - 1) % 2
    for b in range(2):
        scatter_wait(last_slot, STAGE - 2 + b, b)

    plsc.subcore_barrier()

    for kk in range(4):
        r0 = zbase + CHUNK * kk
        pltpu.sync_copy(acc.at[pl.ds(r0, CHUNK)], fb)
        pltpu.sync_copy(fb, out_hbm.at[cid, pl.ds(r0, CHUNK)])
    pltpu.sync_copy(acc.at[pl.ds(zbase + 4 * CHUNK, remr)],
                    fb.at[pl.ds(0, remr)])
    pltpu.sync_copy(fb.at[pl.ds(0, remr)],
                    out_hbm.at[cid, pl.ds(zbase + 4 * CHUNK, remr)])


_sc_spmm = functools.partial(
    pl.kernel,
    out_type=jax.ShapeDtypeStruct((NC, NPAD, D), jnp.float32),
    mesh=plsc.VectorSubcoreMesh(core_axis_name="c", subcore_axis_name="s"),
    scratch_types=[
        pltpu.VMEM((2, STAGE, CHUNK), jnp.int32),
        pltpu.VMEM((2, STAGE, CHUNK), jnp.int32),
        pltpu.VMEM((2, STAGE, CHUNK), jnp.float32),
        pltpu.VMEM((CHUNK, D), jnp.float32),
        pltpu.VMEM((CHUNK, D), jnp.float32),
        pltpu.VMEM_SHARED((NPAD, D), jnp.float32),
        pltpu.SemaphoreType.DMA,
        pltpu.SemaphoreType.DMA,
        pltpu.SemaphoreType.DMA,
        pltpu.SemaphoreType.DMA,
    ],
)(_sc_spmm_body)


BLK = 1000
NBLK = N // BLK


def _tc_linear_body(agg_ref, wt_ref, b_ref, h_ref, stats_ref):
    i = pl.program_id(0)
    a = agg_ref[0] + agg_ref[1]
    h = jnp.dot(a, wt_ref[...], preferred_element_type=jnp.float32) + b_ref[...]
    h_ref[...] = h

    @pl.when(i == 0)
    def _():
        stats_ref[...] = jnp.zeros_like(stats_ref)

    stats_ref[0:1, :] += jnp.sum(h, axis=0, keepdims=True)
    stats_ref[1:2, :] += jnp.sum(h * h, axis=0, keepdims=True)


def _tc_linear(agg2, wt, b2):
    return pl.pallas_call(
        _tc_linear_body,
        grid=(NBLK,),
        in_specs=[
            pl.BlockSpec((NC, BLK, D), lambda i: (0, i, 0)),
            pl.BlockSpec((D, D), lambda i: (0, 0)),
            pl.BlockSpec((1, D), lambda i: (0, 0)),
        ],
        out_specs=[
            pl.BlockSpec((BLK, D), lambda i: (i, 0)),
            pl.BlockSpec((8, D), lambda i: (0, 0)),
        ],
        out_shape=[
            jax.ShapeDtypeStruct((N, D), jnp.float32),
            jax.ShapeDtypeStruct((8, D), jnp.float32),
        ],
    )(agg2, wt, b2)


def _tc_bn_body(h_ref, stats_ref, gamma_ref, beta_ref, out_ref):
    mean = stats_ref[0:1, :] / N
    var = stats_ref[1:2, :] / N - mean * mean
    inv = lax.rsqrt(var + EPS)
    scale = inv * gamma_ref[...]
    shift = beta_ref[...] - mean * scale
    out_ref[...] = h_ref[...] * scale + shift


def _tc_bn(h, stats, gamma2, beta2):
    return pl.pallas_call(
        _tc_bn_body,
        grid=(NBLK,),
        in_specs=[
            pl.BlockSpec((BLK, D), lambda i: (i, 0)),
            pl.BlockSpec((8, D), lambda i: (0, 0)),
            pl.BlockSpec((1, D), lambda i: (0, 0)),
            pl.BlockSpec((1, D), lambda i: (0, 0)),
        ],
        out_specs=pl.BlockSpec((BLK, D), lambda i: (i, 0)),
        out_shape=jax.ShapeDtypeStruct((N, D), jnp.float32),
    )(h, stats, gamma2, beta2)


@jax.jit
def kernel(x, edge_index, edge_weight, W, b, gamma, beta):
    pad = EPAD - E
    pad_row = N + jnp.arange(pad, dtype=jnp.int32) % (NPAD - N)
    pad_col = jnp.arange(pad, dtype=jnp.int32) % N
    row = jnp.concatenate([edge_index[0].astype(jnp.int32), pad_row])
    row = row.reshape(-1, CHUNK)
    col = jnp.concatenate([edge_index[1].astype(jnp.int32), pad_col])
    col = col.reshape(-1, CHUNK)
    ew = jnp.pad(edge_weight, (0, pad)).reshape(-1, CHUNK)
    agg2 = _sc_spmm(x, col, row, ew)
    h, stats = _tc_linear(agg2, W.T, b.reshape(1, D))
    return _tc_bn(h, stats, gamma.reshape(1, D), beta.reshape(1, D))


# speedup vs baseline: 1.0199x; 1.0199x over previous
import functools

import jax
import jax.numpy as jnp
import numpy as np
from jax import lax
from jax.experimental import pallas as pl
from jax.experimental.pallas import tpu as pltpu
from jax.experimental.pallas import tpu_sc as plsc

N = 10000
E = 320000
D = 128
EPS = 1e-5

CHUNK = 128
HALF = CHUNK // 2
NC = 2
NS = 16
NW = NC * NS
KPW = 80
STAGE = 8
NSTAGE = KPW // STAGE
EPAD = NW * KPW * CHUNK
NPAD = 10112
ROWS_PER_TILE = NPAD // NS

def _sc_spmm_body(x_hbm, col_hbm, row_hbm, w_hbm, out_hbm,
                  col_s, row_s, w_s, fb0, fb1,
                  acc, gs0, gs1, ss0, ss1):
    cid = lax.axis_index("c")
    sid = lax.axis_index("s")
    wid = sid * NC + cid
    wstart = wid * KPW
    fbufs = (fb0, fb1)
    gsems = (gs0, gs1)
    ssems = (ss0, ss1)
    fb = fb0

    zero16 = jnp.zeros((16,), jnp.float32)

    def zrow(r, carry):
        for j in range(D // 16):
            fb[r, pl.ds(16 * j, 16)] = zero16
        return carry

    lax.fori_loop(0, CHUNK, zrow, 0)

    zbase = sid * ROWS_PER_TILE
    for kk in range(4):
        pltpu.sync_copy(fb, acc.at[pl.ds(zbase + CHUNK * kk, CHUNK)])
    remr = ROWS_PER_TILE - 4 * CHUNK
    pltpu.sync_copy(fb.at[pl.ds(0, remr)],
                    acc.at[pl.ds(zbase + 4 * CHUNK, remr)])

    plsc.subcore_barrier()

    def load_stage(st, slot):
        off = wstart + STAGE * st
        pltpu.sync_copy(col_hbm.at[pl.ds(off, STAGE)], col_s.at[slot])
        pltpu.sync_copy(row_hbm.at[pl.ds(off, STAGE)], row_s.at[slot])
        pltpu.sync_copy(w_hbm.at[pl.ds(off, STAGE)], w_s.at[slot])

    def gather_start(k, b):
        slot = lax.rem(lax.div(k, STAGE), 2)
        kp = lax.rem(k, STAGE)
        pltpu.async_copy(x_hbm.at[col_s.at[slot, kp]], fbufs[b], gsems[b])

    def gather_wait(k, b):
        slot = lax.rem(lax.div(k, STAGE), 2)
        kp = lax.rem(k, STAGE)
        pltpu.make_async_copy(
            x_hbm.at[col_s.at[slot, kp]], fbufs[b], gsems[b]).wait()

    def scale_inplace(buf, slot, kp):
        def sgroup(g, carry):
            r0 = 16 * g
            wv = w_s[slot, kp, pl.ds(r0, 16)]
            for rp in range(16):
                wr = wv[rp]
                r = r0 + rp
                for j in range(D // 16):
                    sl = pl.ds(16 * j, 16)
                    buf[r, sl] = buf[r, sl] * wr
            return carry

        lax.fori_loop(0, CHUNK // 16, sgroup, 0)

    def scatter_start(slot, kp, b):
        pltpu.async_copy(fbufs[b], acc.at[row_s.at[slot, kp]],
                         ssems[b], add=True)

    def scatter_wait(slot, kp, b):
        pltpu.make_async_copy(fbufs[b], acc.at[row_s.at[slot, kp]],
                              ssems[b]).wait()

    def chunk_body(k, b):
        slot = lax.rem(lax.div(k, STAGE), 2)
        kp = lax.rem(k, STAGE)
        st = lax.div(k, STAGE)

        @pl.when(jnp.logical_and(kp == 0, st + 1 < NSTAGE))
        def _():
            load_stage(st + 1, lax.rem(st + 1, 2))

        @pl.when(k + 1 < KPW)
        def _():
            @pl.when(k >= 1)
            def _():
                scatter_wait(slot, kp, 1 - b)

            gather_start(k + 1, 1 - b)

        gather_wait(k, b)
        scale_inplace(fbufs[b], slot, kp)
        scatter_start(slot, kp, b)

    load_stage(0, 0)
    gather_start(0, 0)

    def pair_body(i, carry):
        chunk_body(2 * i, 0)
        chunk_body(2 * i + 1, 1)
        return carry

    lax.fori_loop(0, KPW // 2, pair_body, 0)

    last_slot = (NSTAGE - 1) % 2
    for b in range(2):
        scatter_wait(last_slot, STAGE - 2 + b, b)

    plsc.subcore_barrier()

    pltpu.sync_copy(acc.at[pl.ds(zbase, ROWS_PER_TILE)],
                    out_hbm.at[cid, pl.ds(zbase, ROWS_PER_TILE)])


_sc_spmm = functools.partial(
    pl.kernel,
    out_type=jax.ShapeDtypeStruct((NC, NPAD, D), jnp.float32),
    mesh=plsc.VectorSubcoreMesh(core_axis_name="c", subcore_axis_name="s"),
    scratch_types=[
        pltpu.VMEM((2, STAGE, CHUNK), jnp.int32),
        pltpu.VMEM((2, STAGE, CHUNK), jnp.int32),
        pltpu.VMEM((2, STAGE, CHUNK), jnp.float32),
        pltpu.VMEM((CHUNK, D), jnp.float32),
        pltpu.VMEM((CHUNK, D), jnp.float32),
        pltpu.VMEM_SHARED((NPAD, D), jnp.float32),
        pltpu.SemaphoreType.DMA,
        pltpu.SemaphoreType.DMA,
        pltpu.SemaphoreType.DMA,
        pltpu.SemaphoreType.DMA,
    ],
)(_sc_spmm_body)


BLK = 1000
NBLK = N // BLK


def _tc_linear_body(agg_ref, wt_ref, b_ref, h_ref, stats_ref):
    i = pl.program_id(0)
    a = agg_ref[0] + agg_ref[1]
    h = jnp.dot(a, wt_ref[...], preferred_element_type=jnp.float32) + b_ref[...]
    h_ref[...] = h

    @pl.when(i == 0)
    def _():
        stats_ref[...] = jnp.zeros_like(stats_ref)

    stats_ref[0:1, :] += jnp.sum(h, axis=0, keepdims=True)
    stats_ref[1:2, :] += jnp.sum(h * h, axis=0, keepdims=True)


def _tc_linear(agg2, wt, b2):
    return pl.pallas_call(
        _tc_linear_body,
        grid=(NBLK,),
        in_specs=[
            pl.BlockSpec((NC, BLK, D), lambda i: (0, i, 0)),
            pl.BlockSpec((D, D), lambda i: (0, 0)),
            pl.BlockSpec((1, D), lambda i: (0, 0)),
        ],
        out_specs=[
            pl.BlockSpec((BLK, D), lambda i: (i, 0)),
            pl.BlockSpec((8, D), lambda i: (0, 0)),
        ],
        out_shape=[
            jax.ShapeDtypeStruct((N, D), jnp.float32),
            jax.ShapeDtypeStruct((8, D), jnp.float32),
        ],
    )(agg2, wt, b2)


def _tc_bn_body(h_ref, stats_ref, gamma_ref, beta_ref, out_ref):
    mean = stats_ref[0:1, :] / N
    var = stats_ref[1:2, :] / N - mean * mean
    inv = lax.rsqrt(var + EPS)
    scale = inv * gamma_ref[...]
    shift = beta_ref[...] - mean * scale
    out_ref[...] = h_ref[...] * scale + shift


def _tc_bn(h, stats, gamma2, beta2):
    return pl.pallas_call(
        _tc_bn_body,
        grid=(NBLK,),
        in_specs=[
            pl.BlockSpec((BLK, D), lambda i: (i, 0)),
            pl.BlockSpec((8, D), lambda i: (0, 0)),
            pl.BlockSpec((1, D), lambda i: (0, 0)),
            pl.BlockSpec((1, D), lambda i: (0, 0)),
        ],
        out_specs=pl.BlockSpec((BLK, D), lambda i: (i, 0)),
        out_shape=jax.ShapeDtypeStruct((N, D), jnp.float32),
    )(h, stats, gamma2, beta2)


@jax.jit
def kernel(x, edge_index, edge_weight, W, b, gamma, beta):
    pad = EPAD - E
    pad_row = N + jnp.arange(pad, dtype=jnp.int32) % (NPAD - N)
    pad_col = jnp.arange(pad, dtype=jnp.int32) % N
    row = jnp.concatenate([edge_index[0].astype(jnp.int32), pad_row])
    row = row.reshape(-1, CHUNK)
    col = jnp.concatenate([edge_index[1].astype(jnp.int32), pad_col])
    col = col.reshape(-1, CHUNK)
    ew = jnp.pad(edge_weight, (0, pad)).reshape(-1, CHUNK)
    agg2 = _sc_spmm(x, col, row, ew)
    h, stats = _tc_linear(agg2, W.T, b.reshape(1, D))
    return _tc_bn(h, stats, gamma.reshape(1, D), beta.reshape(1, D))


# speedup vs baseline: 1.0511x; 1.0306x over previous
import functools

import jax
import jax.numpy as jnp
import numpy as np
from jax import lax
from jax.experimental import pallas as pl
from jax.experimental.pallas import tpu as pltpu
from jax.experimental.pallas import tpu_sc as plsc

N = 10000
E = 320000
D = 128
EPS = 1e-5

CHUNK = 128
HALF = CHUNK // 2
NC = 2
NS = 16
NW = NC * NS
KPW = 80
STAGE = 8
NSTAGE = KPW // STAGE
EPAD = NW * KPW * CHUNK
NPAD = 10112
ROWS_PER_TILE = NPAD // NS

def _sc_spmm_body(x_hbm, col_hbm, row_hbm, w_hbm, out_hbm,
                  col_s, row_s, w_s, fb0, fb1,
                  acc, gs0, gs1, ss0, ss1):
    cid = lax.axis_index("c")
    sid = lax.axis_index("s")
    wid = sid * NC + cid
    wstart = wid * KPW
    fbufs = (fb0, fb1)
    gsems = (gs0, gs1)
    ssems = (ss0, ss1)
    fb = fb0

    zero16 = jnp.zeros((16,), jnp.float32)

    def zrow(r, carry):
        for j in range(D // 16):
            fb[r, pl.ds(16 * j, 16)] = zero16
        return carry

    lax.fori_loop(0, CHUNK, zrow, 0)

    zbase = sid * ROWS_PER_TILE
    for kk in range(4):
        pltpu.sync_copy(fb, acc.at[pl.ds(zbase + CHUNK * kk, CHUNK)])
    remr = ROWS_PER_TILE - 4 * CHUNK
    pltpu.sync_copy(fb.at[pl.ds(0, remr)],
                    acc.at[pl.ds(zbase + 4 * CHUNK, remr)])

    plsc.subcore_barrier()

    def load_stage(st, slot):
        off = wstart + STAGE * st
        pltpu.sync_copy(col_hbm.at[pl.ds(off, STAGE)], col_s.at[slot])
        pltpu.sync_copy(row_hbm.at[pl.ds(off, STAGE)], row_s.at[slot])
        pltpu.sync_copy(w_hbm.at[pl.ds(off, STAGE)], w_s.at[slot])

    def gather_start(k, b):
        slot = lax.rem(lax.div(k, STAGE), 2)
        kp = lax.rem(k, STAGE)
        pltpu.async_copy(x_hbm.at[col_s.at[slot, kp]], fbufs[b], gsems[b])

    def gather_wait(k, b):
        slot = lax.rem(lax.div(k, STAGE), 2)
        kp = lax.rem(k, STAGE)
        pltpu.make_async_copy(
            x_hbm.at[col_s.at[slot, kp]], fbufs[b], gsems[b]).wait()

    def scale_inplace(buf, slot, kp):
        def sgroup(g, carry):
            r0 = 16 * g
            wv = w_s[slot, kp, pl.ds(r0, 16)]
            for rp in range(16):
                wr = wv[rp]
                r = r0 + rp
                for j in range(D // 16):
                    sl = pl.ds(16 * j, 16)
                    buf[r, sl] = buf[r, sl] * wr
            return carry

        lax.fori_loop(0, CHUNK // 16, sgroup, 0)

    def scatter_start(slot, kp, b):
        pltpu.async_copy(fbufs[b], acc.at[row_s.at[slot, kp]],
                         ssems[b], add=True)

    def scatter_wait(slot, kp, b):
        pltpu.make_async_copy(fbufs[b], acc.at[row_s.at[slot, kp]],
                              ssems[b]).wait()

    def chunk_body(k, b):
        slot = lax.rem(lax.div(k, STAGE), 2)
        kp = lax.rem(k, STAGE)
        st = lax.div(k, STAGE)

        @pl.when(jnp.logical_and(kp == 0, st + 1 < NSTAGE))
        def _():
            load_stage(st + 1, lax.rem(st + 1, 2))

        @pl.when(k + 1 < KPW)
        def _():
            @pl.when(k >= 1)
            def _():
                scatter_wait(slot, kp, 1 - b)

            gather_start(k + 1, 1 - b)

        gather_wait(k, b)
        scale_inplace(fbufs[b], slot, kp)
        scatter_start(slot, kp, b)

    load_stage(0, 0)
    gather_start(0, 0)

    def pair_body(i, carry):
        chunk_body(2 * i, 0)
        chunk_body(2 * i + 1, 1)
        return carry

    lax.fori_loop(0, KPW // 2, pair_body, 0)

    last_slot = (NSTAGE - 1) % 2
    for b in range(2):
        scatter_wait(last_slot, STAGE - 2 + b, b)

    plsc.subcore_barrier()

    pltpu.sync_copy(acc.at[pl.ds(zbase, ROWS_PER_TILE)],
                    out_hbm.at[cid, pl.ds(zbase, ROWS_PER_TILE)])


_sc_spmm = functools.partial(
    pl.kernel,
    out_type=jax.ShapeDtypeStruct((NC, NPAD, D), jnp.float32),
    mesh=plsc.VectorSubcoreMesh(core_axis_name="c", subcore_axis_name="s"),
    scratch_types=[
        pltpu.VMEM((2, STAGE, CHUNK), jnp.int32),
        pltpu.VMEM((2, STAGE, CHUNK), jnp.int32),
        pltpu.VMEM((2, STAGE, CHUNK), jnp.float32),
        pltpu.VMEM((CHUNK, D), jnp.float32),
        pltpu.VMEM((CHUNK, D), jnp.float32),
        pltpu.VMEM_SHARED((NPAD, D), jnp.float32),
        pltpu.SemaphoreType.DMA,
        pltpu.SemaphoreType.DMA,
        pltpu.SemaphoreType.DMA,
        pltpu.SemaphoreType.DMA,
    ],
)(_sc_spmm_body)


BLK = 1000
NBLK = N // BLK


def _tc_body(agg_ref, wt_ref, b_ref, gamma_ref, beta_ref, out_ref,
             h_scr, stats_scr):
    i = pl.program_id(0)

    @pl.when(i < NBLK)
    def _():
        a = agg_ref[0] + agg_ref[1]
        h = jnp.dot(a, wt_ref[...], preferred_element_type=jnp.float32)
        h = h + b_ref[...]
        h_scr[pl.ds(i * BLK, BLK), :] = h

        @pl.when(i == 0)
        def _():
            stats_scr[...] = jnp.zeros_like(stats_scr)

        stats_scr[0:1, :] += jnp.sum(h, axis=0, keepdims=True)
        stats_scr[1:2, :] += jnp.sum(h * h, axis=0, keepdims=True)

    @pl.when(i >= NBLK)
    def _():
        j = i - NBLK
        mean = stats_scr[0:1, :] / N
        var = stats_scr[1:2, :] / N - mean * mean
        inv = lax.rsqrt(var + EPS)
        scale = inv * gamma_ref[...]
        shift = beta_ref[...] - mean * scale
        out_ref[...] = h_scr[pl.ds(j * BLK, BLK), :] * scale + shift


def _tc_fused(agg2, wt, b2, gamma2, beta2):
    return pl.pallas_call(
        _tc_body,
        grid=(2 * NBLK,),
        in_specs=[
            pl.BlockSpec((NC, BLK, D),
                         lambda i: (0, jnp.minimum(i, NBLK - 1), 0)),
            pl.BlockSpec((D, D), lambda i: (0, 0)),
            pl.BlockSpec((1, D), lambda i: (0, 0)),
            pl.BlockSpec((1, D), lambda i: (0, 0)),
            pl.BlockSpec((1, D), lambda i: (0, 0)),
        ],
        out_specs=pl.BlockSpec(
            (BLK, D), lambda i: (jnp.maximum(i - NBLK, 0), 0)),
        out_shape=jax.ShapeDtypeStruct((N, D), jnp.float32),
        scratch_shapes=[
            pltpu.VMEM((N, D), jnp.float32),
            pltpu.VMEM((8, D), jnp.float32),
        ],
    )(agg2, wt, b2, gamma2, beta2)


@jax.jit
def kernel(x, edge_index, edge_weight, W, b, gamma, beta):
    pad = EPAD - E
    pad_row = N + jnp.arange(pad, dtype=jnp.int32) % (NPAD - N)
    pad_col = jnp.arange(pad, dtype=jnp.int32) % N
    row = jnp.concatenate([edge_index[0].astype(jnp.int32), pad_row])
    row = row.reshape(-1, CHUNK)
    col = jnp.concatenate([edge_index[1].astype(jnp.int32), pad_col])
    col = col.reshape(-1, CHUNK)
    ew = jnp.pad(edge_weight, (0, pad)).reshape(-1, CHUNK)
    agg2 = _sc_spmm(x, col, row, ew)
    return _tc_fused(agg2, W.T, b.reshape(1, D),
                     gamma.reshape(1, D), beta.reshape(1, D))


# speedup vs baseline: 1.1020x; 1.0484x over previous
import functools

import jax
import jax.numpy as jnp
import numpy as np
from jax import lax
from jax.experimental import pallas as pl
from jax.experimental.pallas import tpu as pltpu
from jax.experimental.pallas import tpu_sc as plsc

N = 10000
E = 320000
D = 128
EPS = 1e-5

CHUNK = 128
HALF = CHUNK // 2
NC = 2
NS = 16
NW = NC * NS
KPW = 80
STAGE = 8
NSTAGE = KPW // STAGE
EPAD = NW * KPW * CHUNK
NPAD = 10112
ROWS_PER_TILE = NPAD // NS

def _sc_spmm_body(x_hbm, col_hbm, row_hbm, w_hbm, out_hbm,
                  col_s, row_s, w_s, fb0, fb1,
                  acc, gs0, gs1, ss0, ss1, ls):
    cid = lax.axis_index("c")
    sid = lax.axis_index("s")
    wid = sid * NC + cid
    wstart = wid * KPW
    fbufs = (fb0, fb1)
    gsems = (gs0, gs1)
    ssems = (ss0, ss1)
    fb = fb0

    def _prologue():
        pass

    zero16 = jnp.zeros((16,), jnp.float32)

    def zrow(r, carry):
        for j in range(D // 16):
            fb[r, pl.ds(16 * j, 16)] = zero16
        return carry

    lax.fori_loop(0, CHUNK, zrow, 0)

    zbase = sid * ROWS_PER_TILE
    for kk in range(4):
        pltpu.sync_copy(fb, acc.at[pl.ds(zbase + CHUNK * kk, CHUNK)])
    remr = ROWS_PER_TILE - 4 * CHUNK
    pltpu.sync_copy(fb.at[pl.ds(0, remr)],
                    acc.at[pl.ds(zbase + 4 * CHUNK, remr)])

    plsc.subcore_barrier()

    def load_stage(st, slot):
        off = wstart + STAGE * st
        pltpu.async_copy(col_hbm.at[pl.ds(off, STAGE)], col_s.at[slot], ls)
        pltpu.async_copy(row_hbm.at[pl.ds(off, STAGE)], row_s.at[slot], ls)
        pltpu.async_copy(w_hbm.at[pl.ds(off, STAGE)], w_s.at[slot], ls)

    def load_stage_wait(st, slot):
        off = wstart + STAGE * st
        pltpu.make_async_copy(
            col_hbm.at[pl.ds(off, STAGE)], col_s.at[slot], ls).wait()
        pltpu.make_async_copy(
            row_hbm.at[pl.ds(off, STAGE)], row_s.at[slot], ls).wait()
        pltpu.make_async_copy(
            w_hbm.at[pl.ds(off, STAGE)], w_s.at[slot], ls).wait()

    def gather_start(k, b):
        slot = lax.rem(lax.div(k, STAGE), 2)
        kp = lax.rem(k, STAGE)
        pltpu.async_copy(x_hbm.at[col_s.at[slot, kp]], fbufs[b], gsems[b])

    def gather_wait(k, b):
        slot = lax.rem(lax.div(k, STAGE), 2)
        kp = lax.rem(k, STAGE)
        pltpu.make_async_copy(
            x_hbm.at[col_s.at[slot, kp]], fbufs[b], gsems[b]).wait()

    def scale_inplace(buf, slot, kp):
        def sgroup(g, carry):
            r0 = 16 * g
            wv = w_s[slot, kp, pl.ds(r0, 16)]
            for rp in range(16):
                wr = wv[rp]
                r = r0 + rp
                for j in range(D // 16):
                    sl = pl.ds(16 * j, 16)
                    buf[r, sl] = buf[r, sl] * wr
            return carry

        lax.fori_loop(0, CHUNK // 16, sgroup, 0)

    def scatter_start(slot, kp, b):
        pltpu.async_copy(fbufs[b], acc.at[row_s.at[slot, kp]],
                         ssems[b], add=True)

    def scatter_wait(slot, kp, b):
        pltpu.make_async_copy(fbufs[b], acc.at[row_s.at[slot, kp]],
                              ssems[b]).wait()

    def chunk_body(k, b):
        slot = lax.rem(lax.div(k, STAGE), 2)
        kp = lax.rem(k, STAGE)
        st = lax.div(k, STAGE)

        @pl.when(jnp.logical_and(kp == 0, st + 1 < NSTAGE))
        def _():
            load_stage(st + 1, lax.rem(st + 1, 2))

        @pl.when(jnp.logical_and(kp == STAGE - 1, st + 1 < NSTAGE))
        def _():
            load_stage_wait(st + 1, lax.rem(st + 1, 2))

        @pl.when(k + 1 < KPW)
        def _():
            @pl.when(k >= 1)
            def _():
                scatter_wait(slot, kp, 1 - b)

            gather_start(k + 1, 1 - b)

        gather_wait(k, b)
        scale_inplace(fbufs[b], slot, kp)
        scatter_start(slot, kp, b)

    load_stage(0, 0)
    load_stage_wait(0, 0)
    gather_start(0, 0)

    def pair_body(i, carry):
        chunk_body(2 * i, 0)
        chunk_body(2 * i + 1, 1)
        return carry

    lax.fori_loop(0, KPW // 2, pair_body, 0)

    last_slot = (NSTAGE - 1) % 2
    for b in range(2):
        scatter_wait(last_slot, STAGE - 2 + b, b)

    plsc.subcore_barrier()

    pltpu.sync_copy(acc.at[pl.ds(zbase, ROWS_PER_TILE)],
                    out_hbm.at[cid, pl.ds(zbase, ROWS_PER_TILE)])


_sc_spmm = functools.partial(
    pl.kernel,
    out_type=jax.ShapeDtypeStruct((NC, NPAD, D), jnp.float32),
    mesh=plsc.VectorSubcoreMesh(core_axis_name="c", subcore_axis_name="s"),
    scratch_types=[
        pltpu.VMEM((2, STAGE, CHUNK), jnp.int32),
        pltpu.VMEM((2, STAGE, CHUNK), jnp.int32),
        pltpu.VMEM((2, STAGE, CHUNK), jnp.float32),
        pltpu.VMEM((CHUNK, D), jnp.float32),
        pltpu.VMEM((CHUNK, D), jnp.float32),
        pltpu.VMEM_SHARED((NPAD, D), jnp.float32),
        pltpu.SemaphoreType.DMA,
        pltpu.SemaphoreType.DMA,
        pltpu.SemaphoreType.DMA,
        pltpu.SemaphoreType.DMA,
        pltpu.SemaphoreType.DMA,
    ],
)(_sc_spmm_body)


BLK = 1000
NBLK = N // BLK


def _tc_body(agg_ref, wt_ref, b_ref, gamma_ref, beta_ref, out_ref,
             h_scr, stats_scr):
    i = pl.program_id(0)

    @pl.when(i < NBLK)
    def _():
        a = agg_ref[0] + agg_ref[1]
        h = jnp.dot(a, wt_ref[...], preferred_element_type=jnp.float32)
        h = h + b_ref[...]
        h_scr[pl.ds(i * BLK, BLK), :] = h

        @pl.when(i == 0)
        def _():
            stats_scr[...] = jnp.zeros_like(stats_scr)

        stats_scr[0:1, :] += jnp.sum(h, axis=0, keepdims=True)
        stats_scr[1:2, :] += jnp.sum(h * h, axis=0, keepdims=True)

    @pl.when(i >= NBLK)
    def _():
        j = i - NBLK
        mean = stats_scr[0:1, :] / N
        var = stats_scr[1:2, :] / N - mean * mean
        inv = lax.rsqrt(var + EPS)
        scale = inv * gamma_ref[...]
        shift = beta_ref[...] - mean * scale
        out_ref[...] = h_scr[pl.ds(j * BLK, BLK), :] * scale + shift


def _tc_fused(agg2, wt, b2, gamma2, beta2):
    return pl.pallas_call(
        _tc_body,
        grid=(2 * NBLK,),
        in_specs=[
            pl.BlockSpec((NC, BLK, D),
                         lambda i: (0, jnp.minimum(i, NBLK - 1), 0)),
            pl.BlockSpec((D, D), lambda i: (0, 0)),
            pl.BlockSpec((1, D), lambda i: (0, 0)),
            pl.BlockSpec((1, D), lambda i: (0, 0)),
            pl.BlockSpec((1, D), lambda i: (0, 0)),
        ],
        out_specs=pl.BlockSpec(
            (BLK, D), lambda i: (jnp.maximum(i - NBLK, 0), 0)),
        out_shape=jax.ShapeDtypeStruct((N, D), jnp.float32),
        scratch_shapes=[
            pltpu.VMEM((N, D), jnp.float32),
            pltpu.VMEM((8, D), jnp.float32),
        ],
    )(agg2, wt, b2, gamma2, beta2)


@jax.jit
def kernel(x, edge_index, edge_weight, W, b, gamma, beta):
    pad = EPAD - E
    pad_row = N + jnp.arange(pad, dtype=jnp.int32) % (NPAD - N)
    pad_col = jnp.arange(pad, dtype=jnp.int32) % N
    row = jnp.concatenate([edge_index[0].astype(jnp.int32), pad_row])
    row = row.reshape(-1, CHUNK)
    col = jnp.concatenate([edge_index[1].astype(jnp.int32), pad_col])
    col = col.reshape(-1, CHUNK)
    ew = jnp.pad(edge_weight, (0, pad)).reshape(-1, CHUNK)
    agg2 = _sc_spmm(x, col, row, ew)
    return _tc_fused(agg2, W.T, b.reshape(1, D),
                     gamma.reshape(1, D), beta.reshape(1, D))


# speedup vs baseline: 1.1297x; 1.0252x over previous
import functools

import jax
import jax.numpy as jnp
import numpy as np
from jax import lax
from jax.experimental import pallas as pl
from jax.experimental.pallas import tpu as pltpu
from jax.experimental.pallas import tpu_sc as plsc

N = 10000
E = 320000
D = 128
EPS = 1e-5

CHUNK = 128
HALF = CHUNK // 2
NC = 2
NS = 16
NW = NC * NS
KPW = 80
STAGE = 8
NSTAGE = KPW // STAGE
EPAD = NW * KPW * CHUNK
NPAD = 10112
ROWS_PER_TILE = NPAD // NS

def _sc_spmm_body(x_hbm, col_hbm, row_hbm, w_hbm, out_hbm,
                  col_s, row_s, w_s, fb0, fb1,
                  acc, gs0, gs1, ss0, ss1, ls):
    cid = lax.axis_index("c")
    sid = lax.axis_index("s")
    wid = sid * NC + cid
    wstart = wid * KPW
    fbufs = (fb0, fb1)
    gsems = (gs0, gs1)
    ssems = (ss0, ss1)
    fb = fb0

    zero16 = jnp.zeros((16,), jnp.float32)

    def zrow(r, carry):
        for j in range(D // 16):
            fb[r, pl.ds(16 * j, 16)] = zero16
        return carry

    lax.fori_loop(0, CHUNK, zrow, 0)

    zbase = sid * ROWS_PER_TILE
    for kk in range(4):
        pltpu.sync_copy(fb, acc.at[pl.ds(zbase + CHUNK * kk, CHUNK)])
    remr = ROWS_PER_TILE - 4 * CHUNK
    pltpu.sync_copy(fb.at[pl.ds(0, remr)],
                    acc.at[pl.ds(zbase + 4 * CHUNK, remr)])

    plsc.subcore_barrier()

    def load_stage(st, slot):
        off = wstart + STAGE * st
        pltpu.async_copy(col_hbm.at[pl.ds(off, STAGE)], col_s.at[slot], ls)
        pltpu.async_copy(row_hbm.at[pl.ds(off, STAGE)], row_s.at[slot], ls)
        pltpu.async_copy(w_hbm.at[pl.ds(off, STAGE)], w_s.at[slot], ls)

    def load_stage_wait(st, slot):
        off = wstart + STAGE * st
        pltpu.make_async_copy(
            col_hbm.at[pl.ds(off, STAGE)], col_s.at[slot], ls).wait()
        pltpu.make_async_copy(
            row_hbm.at[pl.ds(off, STAGE)], row_s.at[slot], ls).wait()
        pltpu.make_async_copy(
            w_hbm.at[pl.ds(off, STAGE)], w_s.at[slot], ls).wait()

    def gather_start(k, b):
        slot = lax.rem(lax.div(k, STAGE), 2)
        kp = lax.rem(k, STAGE)
        pltpu.async_copy(x_hbm.at[col_s.at[slot, kp]], fbufs[b], gsems[b])

    def gather_wait(k, b):
        slot = lax.rem(lax.div(k, STAGE), 2)
        kp = lax.rem(k, STAGE)
        pltpu.make_async_copy(
            x_hbm.at[col_s.at[slot, kp]], fbufs[b], gsems[b]).wait()

    def scale_inplace(buf, slot, kp):
        def sgroup(g, carry):
            r0 = 16 * g
            wv = w_s[slot, kp, pl.ds(r0, 16)]
            for rp in range(16):
                wr = wv[rp]
                r = r0 + rp
                for j in range(D // 16):
                    sl = pl.ds(16 * j, 16)
                    buf[r, sl] = buf[r, sl] * wr
            return carry

        lax.fori_loop(0, CHUNK // 16, sgroup, 0)

    def scatter_start(slot, kp, b):
        pltpu.async_copy(fbufs[b], acc.at[row_s.at[slot, kp]],
                         ssems[b], add=True)

    def scatter_wait(slot, kp, b):
        pltpu.make_async_copy(fbufs[b], acc.at[row_s.at[slot, kp]],
                              ssems[b]).wait()

    def chunk_body(k, b):
        slot = lax.rem(lax.div(k, STAGE), 2)
        kp = lax.rem(k, STAGE)
        st = lax.div(k, STAGE)

        @pl.when(jnp.logical_and(kp == 0, st + 1 < NSTAGE))
        def _():
            load_stage(st + 1, lax.rem(st + 1, 2))

        @pl.when(jnp.logical_and(kp == STAGE - 1, st + 1 < NSTAGE))
        def _():
            load_stage_wait(st + 1, lax.rem(st + 1, 2))

        @pl.when(k + 1 < KPW)
        def _():
            @pl.when(k >= 1)
            def _():
                scatter_wait(slot, kp, 1 - b)

            gather_start(k + 1, 1 - b)

        gather_wait(k, b)
        scale_inplace(fbufs[b], slot, kp)
        scatter_start(slot, kp, b)

    load_stage(0, 0)
    load_stage_wait(0, 0)
    gather_start(0, 0)

    def pair_body(i, carry):
        chunk_body(2 * i, 0)
        chunk_body(2 * i + 1, 1)
        return carry

    lax.fori_loop(0, KPW // 2, pair_body, 0)

    last_slot = (NSTAGE - 1) % 2
    for b in range(2):
        scatter_wait(last_slot, STAGE - 2 + b, b)

    plsc.subcore_barrier()

    pltpu.sync_copy(acc.at[pl.ds(zbase, ROWS_PER_TILE)],
                    out_hbm.at[cid, pl.ds(zbase, ROWS_PER_TILE)])


_sc_spmm = functools.partial(
    pl.kernel,
    out_type=jax.ShapeDtypeStruct((NC, NPAD, D), jnp.float32),
    mesh=plsc.VectorSubcoreMesh(core_axis_name="c", subcore_axis_name="s"),
    scratch_types=[
        pltpu.VMEM((2, STAGE, CHUNK), jnp.int32),
        pltpu.VMEM((2, STAGE, CHUNK), jnp.int32),
        pltpu.VMEM((2, STAGE, CHUNK), jnp.float32),
        pltpu.VMEM((CHUNK, D), jnp.float32),
        pltpu.VMEM((CHUNK, D), jnp.float32),
        pltpu.VMEM_SHARED((NPAD, D), jnp.float32),
        pltpu.SemaphoreType.DMA,
        pltpu.SemaphoreType.DMA,
        pltpu.SemaphoreType.DMA,
        pltpu.SemaphoreType.DMA,
        pltpu.SemaphoreType.DMA,
    ],
)(_sc_spmm_body)


BLK = 2000
NBLK = N // BLK


def _tc_body(agg_ref, wt_ref, b_ref, gamma_ref, beta_ref, out_ref,
             h_scr, stats_scr):
    i = pl.program_id(0)

    @pl.when(i < NBLK)
    def _():
        a = agg_ref[0] + agg_ref[1]
        h = jnp.dot(a, wt_ref[...], preferred_element_type=jnp.float32)
        h = h + b_ref[...]
        h_scr[pl.ds(i * BLK, BLK), :] = h

        @pl.when(i == 0)
        def _():
            stats_scr[...] = jnp.zeros_like(stats_scr)

        stats_scr[0:1, :] += jnp.sum(h, axis=0, keepdims=True)
        stats_scr[1:2, :] += jnp.sum(h * h, axis=0, keepdims=True)

    @pl.when(i >= NBLK)
    def _():
        j = i - NBLK
        mean = stats_scr[0:1, :] / N
        var = stats_scr[1:2, :] / N - mean * mean
        inv = lax.rsqrt(var + EPS)
        scale = inv * gamma_ref[...]
        shift = beta_ref[...] - mean * scale
        out_ref[...] = h_scr[pl.ds(j * BLK, BLK), :] * scale + shift


def _tc_fused(agg2, wt, b2, gamma2, beta2):
    return pl.pallas_call(
        _tc_body,
        grid=(2 * NBLK,),
        in_specs=[
            pl.BlockSpec((NC, BLK, D),
                         lambda i: (0, jnp.minimum(i, NBLK - 1), 0)),
            pl.BlockSpec((D, D), lambda i: (0, 0)),
            pl.BlockSpec((1, D), lambda i: (0, 0)),
            pl.BlockSpec((1, D), lambda i: (0, 0)),
            pl.BlockSpec((1, D), lambda i: (0, 0)),
        ],
        out_specs=pl.BlockSpec(
            (BLK, D), lambda i: (jnp.maximum(i - NBLK, 0), 0)),
        out_shape=jax.ShapeDtypeStruct((N, D), jnp.float32),
        scratch_shapes=[
            pltpu.VMEM((N, D), jnp.float32),
            pltpu.VMEM((8, D), jnp.float32),
        ],
    )(agg2, wt, b2, gamma2, beta2)


@jax.jit
def kernel(x, edge_index, edge_weight, W, b, gamma, beta):
    pad = EPAD - E
    pad_row = N + jnp.arange(pad, dtype=jnp.int32) % (NPAD - N)
    pad_col = jnp.arange(pad, dtype=jnp.int32) % N
    row = jnp.concatenate([edge_index[0].astype(jnp.int32), pad_row])
    row = row.reshape(-1, CHUNK)
    col = jnp.concatenate([edge_index[1].astype(jnp.int32), pad_col])
    col = col.reshape(-1, CHUNK)
    ew = jnp.pad(edge_weight, (0, pad)).reshape(-1, CHUNK)
    agg2 = _sc_spmm(x, col, row, ew)
    return _tc_fused(agg2, W.T, b.reshape(1, D),
                     gamma.reshape(1, D), beta.reshape(1, D))


# speedup vs baseline: 1.1381x; 1.0074x over previous
import functools

import jax
import jax.numpy as jnp
import numpy as np
from jax import lax
from jax.experimental import pallas as pl
from jax.experimental.pallas import tpu as pltpu
from jax.experimental.pallas import tpu_sc as plsc

N = 10000
E = 320000
D = 128
EPS = 1e-5

CHUNK = 128
HALF = CHUNK // 2
NC = 2
NS = 16
NW = NC * NS
KPW = 80
STAGE = 8
NSTAGE = KPW // STAGE
EPAD = NW * KPW * CHUNK
NPAD = 10112
ROWS_PER_TILE = NPAD // NS

def _sc_spmm_body(x_hbm, col_hbm, row_hbm, w_hbm, out_hbm,
                  col_s, row_s, w_s, fb0, fb1,
                  acc, gs0, gs1, ss0, ss1, ls):
    cid = lax.axis_index("c")
    sid = lax.axis_index("s")
    wid = sid * NC + cid
    wstart = wid * KPW
    fbufs = (fb0, fb1)
    gsems = (gs0, gs1)
    ssems = (ss0, ss1)
    fb = fb0

    zero16 = jnp.zeros((16,), jnp.float32)

    def zrow(r, carry):
        for j in range(D // 16):
            fb[r, pl.ds(16 * j, 16)] = zero16
        return carry

    lax.fori_loop(0, CHUNK, zrow, 0)

    zbase = sid * ROWS_PER_TILE
    for kk in range(4):
        pltpu.sync_copy(fb, acc.at[pl.ds(zbase + CHUNK * kk, CHUNK)])
    remr = ROWS_PER_TILE - 4 * CHUNK
    pltpu.sync_copy(fb.at[pl.ds(0, remr)],
                    acc.at[pl.ds(zbase + 4 * CHUNK, remr)])

    plsc.subcore_barrier()

    def load_stage(st, slot):
        off = wstart + STAGE * st
        pltpu.async_copy(col_hbm.at[pl.ds(off, STAGE)], col_s.at[slot], ls)
        pltpu.async_copy(row_hbm.at[pl.ds(off, STAGE)], row_s.at[slot], ls)
        pltpu.async_copy(w_hbm.at[pl.ds(off, STAGE)], w_s.at[slot], ls)

    def load_stage_wait(st, slot):
        off = wstart + STAGE * st
        pltpu.make_async_copy(
            col_hbm.at[pl.ds(off, STAGE)], col_s.at[slot], ls).wait()
        pltpu.make_async_copy(
            row_hbm.at[pl.ds(off, STAGE)], row_s.at[slot], ls).wait()
        pltpu.make_async_copy(
            w_hbm.at[pl.ds(off, STAGE)], w_s.at[slot], ls).wait()

    def gather_start(k, b):
        slot = lax.rem(lax.div(k, STAGE), 2)
        kp = lax.rem(k, STAGE)
        pltpu.async_copy(x_hbm.at[col_s.at[slot, kp]], fbufs[b], gsems[b])

    def gather_wait(k, b):
        slot = lax.rem(lax.div(k, STAGE), 2)
        kp = lax.rem(k, STAGE)
        pltpu.make_async_copy(
            x_hbm.at[col_s.at[slot, kp]], fbufs[b], gsems[b]).wait()

    def scale_inplace(buf, slot, kp):
        def sgroup(g, carry):
            r0 = 16 * g
            wv = w_s[slot, kp, pl.ds(r0, 16)]
            for rp in range(16):
                wr = wv[rp]
                r = r0 + rp
                for j in range(D // 16):
                    sl = pl.ds(16 * j, 16)
                    buf[r, sl] = buf[r, sl] * wr
            return carry

        lax.fori_loop(0, CHUNK // 16, sgroup, 0)

    def scatter_start(slot, kp, b):
        pltpu.async_copy(fbufs[b], acc.at[row_s.at[slot, kp]],
                         ssems[b], add=True)

    def scatter_wait(slot, kp, b):
        pltpu.make_async_copy(fbufs[b], acc.at[row_s.at[slot, kp]],
                              ssems[b]).wait()

    def chunk_body(k, b):
        slot = lax.rem(lax.div(k, STAGE), 2)
        kp = lax.rem(k, STAGE)
        st = lax.div(k, STAGE)

        @pl.when(jnp.logical_and(kp == 0, st + 1 < NSTAGE))
        def _():
            load_stage(st + 1, lax.rem(st + 1, 2))

        @pl.when(jnp.logical_and(kp == STAGE - 1, st + 1 < NSTAGE))
        def _():
            load_stage_wait(st + 1, lax.rem(st + 1, 2))

        @pl.when(k + 1 < KPW)
        def _():
            @pl.when(k >= 1)
            def _():
                scatter_wait(slot, kp, 1 - b)

            gather_start(k + 1, 1 - b)

        gather_wait(k, b)
        scale_inplace(fbufs[b], slot, kp)
        scatter_start(slot, kp, b)

    load_stage(0, 0)
    load_stage_wait(0, 0)
    gather_start(0, 0)

    def pair_body(i, carry):
        chunk_body(2 * i, 0)
        chunk_body(2 * i + 1, 1)
        return carry

    lax.fori_loop(0, KPW // 2, pair_body, 0)

    last_slot = (NSTAGE - 1) % 2
    for b in range(2):
        scatter_wait(last_slot, STAGE - 2 + b, b)

    plsc.subcore_barrier()

    pltpu.sync_copy(acc.at[pl.ds(zbase, ROWS_PER_TILE)],
                    out_hbm.at[cid, pl.ds(zbase, ROWS_PER_TILE)])


_sc_spmm = functools.partial(
    pl.kernel,
    out_type=jax.ShapeDtypeStruct((NC, NPAD, D), jnp.float32),
    mesh=plsc.VectorSubcoreMesh(core_axis_name="c", subcore_axis_name="s"),
    scratch_types=[
        pltpu.VMEM((2, STAGE, CHUNK), jnp.int32),
        pltpu.VMEM((2, STAGE, CHUNK), jnp.int32),
        pltpu.VMEM((2, STAGE, CHUNK), jnp.float32),
        pltpu.VMEM((CHUNK, D), jnp.float32),
        pltpu.VMEM((CHUNK, D), jnp.float32),
        pltpu.VMEM_SHARED((NPAD, D), jnp.float32),
        pltpu.SemaphoreType.DMA,
        pltpu.SemaphoreType.DMA,
        pltpu.SemaphoreType.DMA,
        pltpu.SemaphoreType.DMA,
        pltpu.SemaphoreType.DMA,
    ],
)(_sc_spmm_body)


BLK = 5000
NBLK = N // BLK


def _tc_body(agg_ref, wt_ref, b_ref, gamma_ref, beta_ref, out_ref,
             h_scr, stats_scr):
    i = pl.program_id(0)

    @pl.when(i < NBLK)
    def _():
        a = agg_ref[0] + agg_ref[1]
        h = jnp.dot(a, wt_ref[...], preferred_element_type=jnp.float32)
        h = h + b_ref[...]
        h_scr[pl.ds(i * BLK, BLK), :] = h

        @pl.when(i == 0)
        def _():
            stats_scr[...] = jnp.zeros_like(stats_scr)

        stats_scr[0:1, :] += jnp.sum(h, axis=0, keepdims=True)
        stats_scr[1:2, :] += jnp.sum(h * h, axis=0, keepdims=True)

    @pl.when(i >= NBLK)
    def _():
        j = i - NBLK
        mean = stats_scr[0:1, :] / N
        var = stats_scr[1:2, :] / N - mean * mean
        inv = lax.rsqrt(var + EPS)
        scale = inv * gamma_ref[...]
        shift = beta_ref[...] - mean * scale
        out_ref[...] = h_scr[pl.ds(j * BLK, BLK), :] * scale + shift


def _tc_fused(agg2, wt, b2, gamma2, beta2):
    return pl.pallas_call(
        _tc_body,
        grid=(2 * NBLK,),
        in_specs=[
            pl.BlockSpec((NC, BLK, D),
                         lambda i: (0, jnp.minimum(i, NBLK - 1), 0)),
            pl.BlockSpec((D, D), lambda i: (0, 0)),
            pl.BlockSpec((1, D), lambda i: (0, 0)),
            pl.BlockSpec((1, D), lambda i: (0, 0)),
            pl.BlockSpec((1, D), lambda i: (0, 0)),
        ],
        out_specs=pl.BlockSpec(
            (BLK, D), lambda i: (jnp.maximum(i - NBLK, 0), 0)),
        out_shape=jax.ShapeDtypeStruct((N, D), jnp.float32),
        scratch_shapes=[
            pltpu.VMEM((N, D), jnp.float32),
            pltpu.VMEM((8, D), jnp.float32),
        ],
    )(agg2, wt, b2, gamma2, beta2)


@jax.jit
def kernel(x, edge_index, edge_weight, W, b, gamma, beta):
    pad = EPAD - E
    pad_row = N + jnp.arange(pad, dtype=jnp.int32) % (NPAD - N)
    pad_col = jnp.arange(pad, dtype=jnp.int32) % N
    row = jnp.concatenate([edge_index[0].astype(jnp.int32), pad_row])
    row = row.reshape(-1, CHUNK)
    col = jnp.concatenate([edge_index[1].astype(jnp.int32), pad_col])
    col = col.reshape(-1, CHUNK)
    ew = jnp.pad(edge_weight, (0, pad)).reshape(-1, CHUNK)
    agg2 = _sc_spmm(x, col, row, ew)
    return _tc_fused(agg2, W.T, b.reshape(1, D),
                     gamma.reshape(1, D), beta.reshape(1, D))


# speedup vs baseline: 1.1429x; 1.0043x over previous
import functools

import jax
import jax.numpy as jnp
import numpy as np
from jax import lax
from jax.experimental import pallas as pl
from jax.experimental.pallas import tpu as pltpu
from jax.experimental.pallas import tpu_sc as plsc

N = 10000
E = 320000
D = 128
EPS = 1e-5

CHUNK = 128
HALF = CHUNK // 2
NC = 2
NS = 16
NW = NC * NS
KPW = 80
STAGE = 8
NSTAGE = KPW // STAGE
EPAD = NW * KPW * CHUNK
NPAD = 10112
ROWS_PER_TILE = NPAD // NS

def _sc_spmm_body(x_hbm, col_hbm, row_hbm, w_hbm, out_hbm,
                  col_s, row_s, w_s, fb0, fb1,
                  acc, gs0, gs1, ss0, ss1, ls):
    cid = lax.axis_index("c")
    sid = lax.axis_index("s")
    wid = sid * NC + cid
    wstart = wid * KPW
    fbufs = (fb0, fb1)
    gsems = (gs0, gs1)
    ssems = (ss0, ss1)
    fb = fb0

    zero16 = jnp.zeros((16,), jnp.float32)

    def zrow(r, carry):
        for j in range(D // 16):
            fb[r, pl.ds(16 * j, 16)] = zero16
        return carry

    lax.fori_loop(0, CHUNK, zrow, 0)

    zbase = sid * ROWS_PER_TILE
    for kk in range(4):
        pltpu.sync_copy(fb, acc.at[pl.ds(zbase + CHUNK * kk, CHUNK)])
    remr = ROWS_PER_TILE - 4 * CHUNK
    pltpu.sync_copy(fb.at[pl.ds(0, remr)],
                    acc.at[pl.ds(zbase + 4 * CHUNK, remr)])

    plsc.subcore_barrier()

    def load_stage(st, slot):
        off = wstart + STAGE * st
        pltpu.async_copy(col_hbm.at[pl.ds(off, STAGE)], col_s.at[slot], ls)
        pltpu.async_copy(row_hbm.at[pl.ds(off, STAGE)], row_s.at[slot], ls)
        pltpu.async_copy(w_hbm.at[pl.ds(off, STAGE)], w_s.at[slot], ls)

    def load_stage_wait(st, slot):
        off = wstart + STAGE * st
        pltpu.make_async_copy(
            col_hbm.at[pl.ds(off, STAGE)], col_s.at[slot], ls).wait()
        pltpu.make_async_copy(
            row_hbm.at[pl.ds(off, STAGE)], row_s.at[slot], ls).wait()
        pltpu.make_async_copy(
            w_hbm.at[pl.ds(off, STAGE)], w_s.at[slot], ls).wait()

    def gather_start(k, b):
        slot = lax.rem(lax.div(k, STAGE), 2)
        kp = lax.rem(k, STAGE)
        pltpu.async_copy(x_hbm.at[col_s.at[slot, kp]], fbufs[b], gsems[b])

    def gather_wait(k, b):
        slot = lax.rem(lax.div(k, STAGE), 2)
        kp = lax.rem(k, STAGE)
        pltpu.make_async_copy(
            x_hbm.at[col_s.at[slot, kp]], fbufs[b], gsems[b]).wait()

    def scale_inplace(buf, slot, kp):
        @plsc.parallel_loop(0, CHUNK // 16, unroll=2)
        def sgroup(g):
            r0 = 16 * g
            wv = w_s[slot, kp, pl.ds(r0, 16)]
            for rp in range(16):
                wr = wv[rp]
                r = r0 + rp
                for j in range(D // 16):
                    sl = pl.ds(16 * j, 16)
                    buf[r, sl] = buf[r, sl] * wr

    def scatter_start(slot, kp, b):
        pltpu.async_copy(fbufs[b], acc.at[row_s.at[slot, kp]],
                         ssems[b], add=True)

    def scatter_wait(slot, kp, b):
        pltpu.make_async_copy(fbufs[b], acc.at[row_s.at[slot, kp]],
                              ssems[b]).wait()

    def chunk_body(k, b):
        slot = lax.rem(lax.div(k, STAGE), 2)
        kp = lax.rem(k, STAGE)
        st = lax.div(k, STAGE)

        @pl.when(jnp.logical_and(kp == 0, st + 1 < NSTAGE))
        def _():
            load_stage(st + 1, lax.rem(st + 1, 2))

        @pl.when(jnp.logical_and(kp == STAGE - 1, st + 1 < NSTAGE))
        def _():
            load_stage_wait(st + 1, lax.rem(st + 1, 2))

        @pl.when(k + 1 < KPW)
        def _():
            @pl.when(k >= 1)
            def _():
                scatter_wait(slot, kp, 1 - b)

            gather_start(k + 1, 1 - b)

        gather_wait(k, b)
        scale_inplace(fbufs[b], slot, kp)
        scatter_start(slot, kp, b)

    load_stage(0, 0)
    load_stage_wait(0, 0)
    gather_start(0, 0)

    def pair_body(i, carry):
        chunk_body(2 * i, 0)
        chunk_body(2 * i + 1, 1)
        return carry

    lax.fori_loop(0, KPW // 2, pair_body, 0)

    last_slot = (NSTAGE - 1) % 2
    for b in range(2):
        scatter_wait(last_slot, STAGE - 2 + b, b)

    plsc.subcore_barrier()

    pltpu.sync_copy(acc.at[pl.ds(zbase, ROWS_PER_TILE)],
                    out_hbm.at[cid, pl.ds(zbase, ROWS_PER_TILE)])


_sc_spmm = functools.partial(
    pl.kernel,
    out_type=jax.ShapeDtypeStruct((NC, NPAD, D), jnp.float32),
    mesh=plsc.VectorSubcoreMesh(core_axis_name="c", subcore_axis_name="s"),
    scratch_types=[
        pltpu.VMEM((2, STAGE, CHUNK), jnp.int32),
        pltpu.VMEM((2, STAGE, CHUNK), jnp.int32),
        pltpu.VMEM((2, STAGE, CHUNK), jnp.float32),
        pltpu.VMEM((CHUNK, D), jnp.float32),
        pltpu.VMEM((CHUNK, D), jnp.float32),
        pltpu.VMEM_SHARED((NPAD, D), jnp.float32),
        pltpu.SemaphoreType.DMA,
        pltpu.SemaphoreType.DMA,
        pltpu.SemaphoreType.DMA,
        pltpu.SemaphoreType.DMA,
        pltpu.SemaphoreType.DMA,
    ],
)(_sc_spmm_body)


BLK = 5000
NBLK = N // BLK


def _tc_body(agg_ref, wt_ref, b_ref, gamma_ref, beta_ref, out_ref,
             h_scr, stats_scr):
    i = pl.program_id(0)

    @pl.when(i < NBLK)
    def _():
        a = agg_ref[0] + agg_ref[1]
        h = jnp.dot(a, wt_ref[...], preferred_element_type=jnp.float32)
        h = h + b_ref[...]
        h_scr[pl.ds(i * BLK, BLK), :] = h

        @pl.when(i == 0)
        def _():
            stats_scr[...] = jnp.zeros_like(stats_scr)

        stats_scr[0:1, :] += jnp.sum(h, axis=0, keepdims=True)
        stats_scr[1:2, :] += jnp.sum(h * h, axis=0, keepdims=True)

    @pl.when(i >= NBLK)
    def _():
        j = i - NBLK
        mean = stats_scr[0:1, :] / N
        var = stats_scr[1:2, :] / N - mean * mean
        inv = lax.rsqrt(var + EPS)
        scale = inv * gamma_ref[...]
        shift = beta_ref[...] - mean * scale
        out_ref[...] = h_scr[pl.ds(j * BLK, BLK), :] * scale + shift


def _tc_fused(agg2, wt, b2, gamma2, beta2):
    return pl.pallas_call(
        _tc_body,
        grid=(2 * NBLK,),
        in_specs=[
            pl.BlockSpec((NC, BLK, D),
                         lambda i: (0, jnp.minimum(i, NBLK - 1), 0)),
            pl.BlockSpec((D, D), lambda i: (0, 0)),
            pl.BlockSpec((1, D), lambda i: (0, 0)),
            pl.BlockSpec((1, D), lambda i: (0, 0)),
            pl.BlockSpec((1, D), lambda i: (0, 0)),
        ],
        out_specs=pl.BlockSpec(
            (BLK, D), lambda i: (jnp.maximum(i - NBLK, 0), 0)),
        out_shape=jax.ShapeDtypeStruct((N, D), jnp.float32),
        scratch_shapes=[
            pltpu.VMEM((N, D), jnp.float32),
            pltpu.VMEM((8, D), jnp.float32),
        ],
    )(agg2, wt, b2, gamma2, beta2)


@jax.jit
def kernel(x, edge_index, edge_weight, W, b, gamma, beta):
    pad = EPAD - E
    pad_row = N + jnp.arange(pad, dtype=jnp.int32) % (NPAD - N)
    pad_col = jnp.arange(pad, dtype=jnp.int32) % N
    row = jnp.concatenate([edge_index[0].astype(jnp.int32), pad_row])
    row = row.reshape(-1, CHUNK)
    col = jnp.concatenate([edge_index[1].astype(jnp.int32), pad_col])
    col = col.reshape(-1, CHUNK)
    ew = jnp.pad(edge_weight, (0, pad)).reshape(-1, CHUNK)
    agg2 = _sc_spmm(x, col, row, ew)
    return _tc_fused(agg2, W.T, b.reshape(1, D),
                     gamma.reshape(1, D), beta.reshape(1, D))
